# baseline (speedup 1.0000x reference)
import functools

import jax
import jax.numpy as jnp
from jax import lax
from jax.experimental import pallas as pl
from jax.experimental.pallas import tpu as pltpu
from jax.experimental.pallas import tpu_sc as plsc

VOCAB = 1000000
DIM = 32
BATCH = 4096
HIST = 200

NUM_CORES = 2
NUM_SUBCORES = 16
NW = NUM_CORES * NUM_SUBCORES
BCOLS = BATCH // NW
RT = DIM // 8


KH = 5
NG = HIST // KH
GROWS = KH * BCOLS


DCHUNK = 320
NCHUNK = VOCAB // DCHUNK
OROWS = DCHUNK // 4


def _make_detile():
  mesh = plsc.VectorSubcoreMesh(core_axis_name="c", subcore_axis_name="s")

  @functools.partial(
      pl.kernel,
      mesh=mesh,
      out_type=jax.ShapeDtypeStruct((VOCAB // 4, 4 * DIM), jnp.float32),
      scratch_types=[
          pltpu.VMEM((DCHUNK, DIM), jnp.float32),
          pltpu.VMEM((OROWS, 4 * DIM), jnp.float32),
      ],
      compiler_params=pltpu.CompilerParams(
          use_tc_tiling_on_sc=True, needs_layout_passes=False
      ),
  )
  def detile_kernel(tab_hbm, out_hbm, slab_v, pack_v):
    wid = lax.axis_index("s") * NUM_CORES + lax.axis_index("c")

    def chunk_body(t, carry):
      cid = wid + NW * t

      @pl.when(cid < NCHUNK)
      def _():
        pltpu.sync_copy(tab_hbm.at[pl.ds(cid * DCHUNK, DCHUNK)], slab_v)
        for q in range(4):

          def rstep(r4, c, q=q):
            row = r4 * 4 + q
            pack_v[r4, pl.ds(q * DIM, 16)] = slab_v[row, pl.ds(0, 16)]
            pack_v[r4, pl.ds(q * DIM + 16, 16)] = slab_v[row, pl.ds(16, 16)]
            return c

          lax.fori_loop(0, OROWS, rstep, 0)
        pltpu.sync_copy(pack_v, out_hbm.at[pl.ds(cid * OROWS, OROWS)])

      return carry

    lax.fori_loop(0, (NCHUNK + NW - 1) // NW, chunk_body, 0)

  return detile_kernel


_detile = _make_detile()


def _make_gather():
  mesh = plsc.VectorSubcoreMesh(core_axis_name="c", subcore_axis_name="s")

  @functools.partial(
      pl.kernel,
      mesh=mesh,
      out_type=jax.ShapeDtypeStruct((HIST, RT, NW, 8, BCOLS), jnp.float32),
      scratch_types=[
          pltpu.VMEM((HIST * BCOLS,), jnp.int32),
          pltpu.VMEM((2, GROWS, DIM), jnp.float32),
          pltpu.VMEM((2, KH, DIM, BCOLS + 1), jnp.float32),
          pltpu.SemaphoreType.DMA,
          pltpu.SemaphoreType.DMA,
          pltpu.SemaphoreType.DMA,
          pltpu.SemaphoreType.DMA,
      ],
      compiler_params=pltpu.CompilerParams(
          use_tc_tiling_on_sc=False, needs_layout_passes=False
      ),
  )
  def gather_kernel(idx_hbm, table_hbm, out_hbm, idx_v, rows_v, blk_v,
                    gs0, gs1, ws0, ws1):
    gs = (gs0, gs1)
    ws = (ws0, ws1)
    wid = lax.axis_index("s") * NUM_CORES + lax.axis_index("c")
    pltpu.sync_copy(idx_hbm.at[wid], idx_v)

    def gather_start(g, b):
      pltpu.async_copy(
          table_hbm.at[idx_v.at[pl.ds(g * GROWS, GROWS)]],
          rows_v.at[b], gs[b],
      )

    def gather_wait(b):
      pltpu.make_async_copy(
          table_hbm.at[pl.ds(0, GROWS)], rows_v.at[b], gs[b]
      ).wait()

    def write_start(g, b):
      for rt in range(RT):
        pltpu.async_copy(
            blk_v.at[b, :, pl.ds(rt * 8, 8), pl.ds(0, BCOLS)],
            out_hbm.at[pl.ds(g * KH, KH), rt, wid],
            ws[b],
        )

    def write_wait(b):
      for rt in range(RT):
        pltpu.make_async_copy(
            blk_v.at[b, :, pl.ds(rt * 8, 8), pl.ds(0, BCOLS)],
            out_hbm.at[pl.ds(0, KH), rt, wid],
            ws[b],
        ).wait()

    dvec_lo = lax.iota(jnp.int32, 16)
    dvec_hi = dvec_lo + 16
    CUNROLL = 8

    def transpose(b):
      rows_b = rows_v.at[b]
      blk_b = blk_v.at[b]
      for k in range(KH):
        kvec = jnp.full((16,), k, jnp.int32)

        def cstep(ci, carry, k=k, kvec=kvec):
          for j in range(CUNROLL):
            c = ci * CUNROLL + j
            row = k * BCOLS + c
            lo = rows_b[row, pl.ds(0, 16)]
            hi = rows_b[row, pl.ds(16, 16)]
            cvec = jnp.full((16,), 0, jnp.int32) + c
            plsc.store_scatter(blk_b, [kvec, dvec_lo, cvec], lo)
            plsc.store_scatter(blk_b, [kvec, dvec_hi, cvec], hi)
          return carry

        lax.fori_loop(0, BCOLS // CUNROLL, cstep, 0)

    gather_start(0, 0)

    def step(i, carry):
      g0 = 2 * i
      gather_wait(0)
      gather_start(g0 + 1, 1)

      @pl.when(i > 0)
      def _():
        write_wait(0)

      transpose(0)
      write_start(g0, 0)
      gather_wait(1)

      @pl.when(i < NG // 2 - 1)
      def _():
        gather_start(g0 + 2, 0)

      @pl.when(i > 0)
      def _():
        write_wait(1)

      transpose(1)
      write_start(g0 + 1, 1)
      return carry

    lax.fori_loop(0, NG // 2, step, 0)
    write_wait(0)
    write_wait(1)

  return gather_kernel


_gather = _make_gather()


@jax.jit
def kernel(inputs, table):
  idx_t = inputs.T.astype(jnp.int32)
  idx_w = (
      idx_t.reshape(HIST, NW, BCOLS).transpose(1, 0, 2).reshape(NW, HIST * BCOLS)
  )
  t_lin = _detile(table).reshape(VOCAB, DIM)
  out = _gather(idx_w, t_lin)
  t = out.transpose(2, 4, 0, 1, 3)
  return t.reshape(BATCH, HIST, DIM)


# speedup vs baseline: 1.4985x; 1.4985x over previous
import functools

import jax
import jax.numpy as jnp
from jax import lax
from jax.experimental import pallas as pl
from jax.experimental.pallas import tpu as pltpu
from jax.experimental.pallas import tpu_sc as plsc

VOCAB = 1000000
DIM = 32
BATCH = 4096
HIST = 200

NUM_CORES = 2
NUM_SUBCORES = 16
NW = NUM_CORES * NUM_SUBCORES
BCOLS = BATCH // NW
RT = DIM // 8


KH = 5
NG = HIST // KH
GROWS = KH * BCOLS


def _make_gather():
  mesh = plsc.VectorSubcoreMesh(core_axis_name="c", subcore_axis_name="s")

  @functools.partial(
      pl.kernel,
      mesh=mesh,
      out_type=jax.ShapeDtypeStruct((HIST, RT, NW, 8, BCOLS), jnp.float32),
      scratch_types=[
          pltpu.VMEM((HIST * BCOLS,), jnp.int32),
          pltpu.VMEM((2, GROWS, DIM), jnp.float32),
          pltpu.VMEM((2, KH, DIM, BCOLS + 1), jnp.float32),
          pltpu.SemaphoreType.DMA,
          pltpu.SemaphoreType.DMA,
          pltpu.SemaphoreType.DMA,
          pltpu.SemaphoreType.DMA,
      ],
      compiler_params=pltpu.CompilerParams(
          use_tc_tiling_on_sc=False, needs_layout_passes=False
      ),
  )
  def gather_kernel(idx_hbm, table_hbm, out_hbm, idx_v, rows_v, blk_v,
                    gs0, gs1, ws0, ws1):
    gs = (gs0, gs1)
    ws = (ws0, ws1)
    wid = lax.axis_index("s") * NUM_CORES + lax.axis_index("c")
    pltpu.sync_copy(idx_hbm.at[wid], idx_v)

    def gather_start(g, b):
      pltpu.async_copy(
          table_hbm.at[idx_v.at[pl.ds(g * GROWS, GROWS)]],
          rows_v.at[b], gs[b],
      )

    def gather_wait(b):
      pltpu.make_async_copy(
          table_hbm.at[pl.ds(0, GROWS)], rows_v.at[b], gs[b]
      ).wait()

    def write_start(g, b):
      for rt in range(RT):
        pltpu.async_copy(
            blk_v.at[b, :, pl.ds(rt * 8, 8), pl.ds(0, BCOLS)],
            out_hbm.at[pl.ds(g * KH, KH), rt, wid],
            ws[b],
        )

    def write_wait(b):
      for rt in range(RT):
        pltpu.make_async_copy(
            blk_v.at[b, :, pl.ds(rt * 8, 8), pl.ds(0, BCOLS)],
            out_hbm.at[pl.ds(0, KH), rt, wid],
            ws[b],
        ).wait()

    dvec_lo = lax.iota(jnp.int32, 16)
    dvec_hi = dvec_lo + 16
    CUNROLL = 8

    def transpose(b):
      rows_b = rows_v.at[b]
      blk_b = blk_v.at[b]
      for k in range(KH):
        kvec = jnp.full((16,), k, jnp.int32)

        def cstep(ci, carry, k=k, kvec=kvec):
          for j in range(CUNROLL):
            c = ci * CUNROLL + j
            row = k * BCOLS + c
            lo = rows_b[row, pl.ds(0, 16)]
            hi = rows_b[row, pl.ds(16, 16)]
            cvec = jnp.full((16,), 0, jnp.int32) + c
            plsc.store_scatter(blk_b, [kvec, dvec_lo, cvec], lo)
            plsc.store_scatter(blk_b, [kvec, dvec_hi, cvec], hi)
          return carry

        lax.fori_loop(0, BCOLS // CUNROLL, cstep, 0)

    gather_start(0, 0)

    def step(i, carry):
      g0 = 2 * i
      gather_wait(0)
      gather_start(g0 + 1, 1)

      @pl.when(i > 0)
      def _():
        write_wait(0)

      transpose(0)
      write_start(g0, 0)
      gather_wait(1)

      @pl.when(i < NG // 2 - 1)
      def _():
        gather_start(g0 + 2, 0)

      @pl.when(i > 0)
      def _():
        write_wait(1)

      transpose(1)
      write_start(g0 + 1, 1)
      return carry

    lax.fori_loop(0, NG // 2, step, 0)
    write_wait(0)
    write_wait(1)

  return gather_kernel


_gather = _make_gather()


@jax.jit
def kernel(inputs, table):
  idx_t = inputs.T.astype(jnp.int32)
  idx_w = (
      idx_t.reshape(HIST, NW, BCOLS).transpose(1, 0, 2).reshape(NW, HIST * BCOLS)
  )
  out = _gather(idx_w, table)
  t = out.transpose(2, 4, 0, 1, 3)
  return t.reshape(BATCH, HIST, DIM)
